# baseline (speedup 1.0000x reference)
import functools
import jax
import jax.numpy as jnp
from jax import lax
from jax.experimental import pallas as pl
from jax.experimental.pallas import tpu as pltpu
from jax.experimental.pallas import tpu_sc as plsc

H = 768
L = 16
NC, NS = 2, 16
NW = NC * NS
NBUF = 4
U = 8
EPS = 1e-12


def _rsqrt(x):
    y = plsc.bitcast(jnp.int32(0x5F3759DF) - (plsc.bitcast(x, jnp.int32) >> 1),
                     jnp.float32)
    for _ in range(3):
        y = y * (1.5 - 0.5 * x * y * y)
    return y


def _sc_body(nbatch, pw, ids_hbm, tts_hbm, word_hbm, pos_hbm, type_hbm,
             lnw_hbm, lnb_hbm, out_hbm, wbuf, ptT, xT, wT, bT, tb, idxall,
             ttall, gsem, osem):
    wid = lax.axis_index("c") * NS + lax.axis_index("s")
    p0 = wid * pw
    seq = NW * pw
    lanes = lax.iota(jnp.int32, L)
    zerov = jnp.zeros((L,), jnp.int32)
    onev = jnp.full((L,), 1, jnp.int32)

    pltpu.sync_copy(ids_hbm.at[:, pl.ds(p0, pw)], idxall)
    pltpu.sync_copy(tts_hbm.at[:, pl.ds(p0, pw)], ttall)
    pltpu.sync_copy(type_hbm, tb.at[pl.ds(0, 2)])
    pltpu.sync_copy(pos_hbm.at[pl.ds(p0, pw)], wbuf.at[0])
    pltpu.sync_copy(lnw_hbm, tb.at[2])
    pltpu.sync_copy(lnb_hbm, tb.at[3])

    def build_body(h, c):
        hv = jnp.full((L,), h, jnp.int32)
        pv = plsc.load_gather(wbuf.at[0], [lanes, hv])
        t0 = plsc.load_gather(tb, [zerov, hv])
        t1 = plsc.load_gather(tb, [onev, hv])
        ptT[0, h, :] = pv + t0
        ptT[1, h, :] = pv + t1
        wT[h, :] = plsc.load_gather(tb, [jnp.full((L,), 2, jnp.int32), hv])
        bT[h, :] = plsc.load_gather(tb, [jnp.full((L,), 3, jnp.int32), hv])
        return c

    lax.fori_loop(0, H, build_body, 0, unroll=False)

    def issue_gather(b, s):
        idxv = idxall[b, :]
        pltpu.async_copy(word_hbm.at[idxv], wbuf.at[s], gsem.at[s])

    def wait_gather(s):
        pltpu.make_async_copy(word_hbm.at[pl.ds(0, pw)], wbuf.at[s],
                              gsem.at[s]).wait()

    def wait_out(s):
        pltpu.make_async_copy(wbuf.at[s], out_hbm.at[pl.ds(0, pw)],
                              osem.at[s]).wait()

    def compute_chunk(b, s):
        ttv = ttall[b, :]
        wslot = wbuf.at[s]

        def p1_body(hb, accs):
            a = list(accs)
            for u in range(U):
                h = hb * U + u
                hv = jnp.full((L,), h, jnp.int32)
                g = plsc.load_gather(wslot, [lanes, hv])
                pt = plsc.load_gather(ptT, [ttv, hv, lanes])
                x = g + pt
                xT[h, :] = x
                a[u % 4] = a[u % 4] + x
                a[4 + u % 4] = a[4 + u % 4] + x * x
            return tuple(a)

        z = jnp.zeros((L,), jnp.float32)
        accs = lax.fori_loop(0, H // U, p1_body, (z,) * 8, unroll=False)
        sacc = (accs[0] + accs[1]) + (accs[2] + accs[3])
        qacc = (accs[4] + accs[5]) + (accs[6] + accs[7])
        mu = sacc * (1.0 / H)
        var = qacc * (1.0 / H) - mu * mu
        rs = _rsqrt(var + EPS)

        def p2_body(hb, c):
            for u in range(U):
                h = hb * U + u
                hv = jnp.full((L,), h, jnp.int32)
                y = (xT[h, :] - mu) * (rs * wT[h, :]) + bT[h, :]
                plsc.store_scatter(wslot, [lanes, hv], y)
            return c

        lax.fori_loop(0, H // U, p2_body, 0, unroll=False)
        pltpu.async_copy(wslot, out_hbm.at[pl.ds(b * seq + p0, pw)],
                         osem.at[s])

    issue_gather(0, 0)
    issue_gather(1, 1)

    ngroup = nbatch // NBUF

    def group_body(g, carry):
        for k in range(NBUF):
            b = g * NBUF + k
            if k < 2:
                s2 = k + 2

                @pl.when(g > 0)
                def _():
                    wait_out(s2)

                issue_gather(b + 2, s2)
            else:
                s2 = k - 2

                @pl.when(g < ngroup - 1)
                def _():
                    wait_out(s2)
                    issue_gather(b + 2, s2)

            wait_gather(k)
            compute_chunk(b, k)
        return carry

    lax.fori_loop(0, ngroup, group_body, 0, unroll=False)

    for s in range(NBUF):
        wait_out(s)


def kernel(input_ids, token_type_ids, word_emb, pos_emb, type_emb, ln_w, ln_b):
    b, s = input_ids.shape
    h = word_emb.shape[1]
    assert h == H and s % NW == 0 and b % NBUF == 0
    pw = s // NW
    ids = input_ids.astype(jnp.int32)
    tts = token_type_ids.astype(jnp.int32)

    mesh = plsc.VectorSubcoreMesh(core_axis_name="c", subcore_axis_name="s",
                                  num_cores=NC, num_subcores=NS)
    run = pl.kernel(
        functools.partial(_sc_body, b, pw),
        out_type=jax.ShapeDtypeStruct((b * s, h), jnp.float32),
        mesh=mesh,
        compiler_params=pltpu.CompilerParams(needs_layout_passes=False,
                                             use_tc_tiling_on_sc=False),
        scratch_types=[
            pltpu.VMEM((NBUF, pw, h), jnp.float32),
            pltpu.VMEM((2, h, L), jnp.float32),
            pltpu.VMEM((h, L), jnp.float32),
            pltpu.VMEM((h, L), jnp.float32),
            pltpu.VMEM((h, L), jnp.float32),
            pltpu.VMEM((4, h), jnp.float32),
            pltpu.VMEM((b, pw), jnp.int32),
            pltpu.VMEM((b, pw), jnp.int32),
            pltpu.SemaphoreType.DMA((NBUF,)),
            pltpu.SemaphoreType.DMA((NBUF,)),
        ],
    )
    out = run(ids, tts, word_emb, pos_emb, type_emb, ln_w, ln_b)
    return out.reshape(b, s, h)


# speedup vs baseline: 4.6498x; 4.6498x over previous
import functools
import jax
import jax.numpy as jnp
from jax import lax
from jax.experimental import pallas as pl
from jax.experimental.pallas import tpu as pltpu
from jax.experimental.pallas import tpu_sc as plsc

H = 768
NC, NS = 2, 16
NW = NC * NS
K = 32
NBUF = 4
EPS = 1e-12


def _sc_gather_body(ntok, ids_hbm, word_hbm, out_hbm, bufs, idxall, gsem,
                    osem):
    wid = lax.axis_index("c") * NS + lax.axis_index("s")
    tok0 = wid * ntok
    nchunk = ntok // K

    pltpu.sync_copy(ids_hbm.at[pl.ds(tok0, ntok)], idxall)

    def issue_gather(c, s):
        pltpu.async_copy(word_hbm.at[idxall.at[pl.ds(c * K, K)]], bufs.at[s],
                         gsem.at[s])

    def wait_gather(s):
        pltpu.make_async_copy(word_hbm.at[pl.ds(0, K)], bufs.at[s],
                              gsem.at[s]).wait()

    def wait_out(s):
        pltpu.make_async_copy(bufs.at[s], out_hbm.at[pl.ds(0, K)],
                              osem.at[s]).wait()

    issue_gather(0, 0)
    issue_gather(1, 1)

    ngroup = nchunk // NBUF

    def group_body(g, carry):
        for k in range(NBUF):
            c = g * NBUF + k
            if k < 2:
                s2 = k + 2

                @pl.when(g > 0)
                def _():
                    wait_out(s2)

                issue_gather(c + 2, s2)
            else:
                s2 = k - 2

                @pl.when(g < ngroup - 1)
                def _():
                    wait_out(s2)
                    issue_gather(c + 2, s2)

            wait_gather(k)
            pltpu.async_copy(bufs.at[k], out_hbm.at[pl.ds(tok0 + c * K, K)],
                             osem.at[k])
        return carry

    lax.fori_loop(0, ngroup, group_body, 0, unroll=False)

    for s in range(NBUF):
        wait_out(s)


def _sc_gather(ids, word_emb):
    n = ids.shape[0]
    ntok = n // NW
    mesh = plsc.VectorSubcoreMesh(core_axis_name="c", subcore_axis_name="s",
                                  num_cores=NC, num_subcores=NS)
    return pl.kernel(
        functools.partial(_sc_gather_body, ntok),
        out_type=jax.ShapeDtypeStruct((n, H), jnp.float32),
        mesh=mesh,
        compiler_params=pltpu.CompilerParams(needs_layout_passes=False,
                                             use_tc_tiling_on_sc=False),
        scratch_types=[
            pltpu.VMEM((NBUF, K, H), jnp.float32),
            pltpu.VMEM((ntok,), jnp.int32),
            pltpu.SemaphoreType.DMA((NBUF,)),
            pltpu.SemaphoreType.DMA((NBUF,)),
        ],
    )(ids, word_emb)


def _tc_ln_body(g_ref, tt_ref, pos_ref, type_ref, w_ref, b_ref, o_ref):
    tsel = jnp.where(tt_ref[0] == 1,
                     type_ref[1, :][None, :], type_ref[0, :][None, :])
    x = g_ref[0] + pos_ref[...] + tsel
    mu = jnp.mean(x, axis=-1, keepdims=True)
    xc = x - mu
    var = jnp.mean(xc * xc, axis=-1, keepdims=True)
    y = xc * lax.rsqrt(var + EPS) * w_ref[0][None, :] + b_ref[0][None, :]
    o_ref[0] = y


def _tc_ln(gath, tts, pos_emb, type_emb, ln_w, ln_b):
    b, s = tts.shape
    g3 = gath.reshape(b, s, H)
    tt3 = tts.reshape(b, s, 1)
    return pl.pallas_call(
        _tc_ln_body,
        grid=(b,),
        in_specs=[
            pl.BlockSpec((1, s, H), lambda i: (i, 0, 0)),
            pl.BlockSpec((1, s, 1), lambda i: (i, 0, 0)),
            pl.BlockSpec((s, H), lambda i: (0, 0)),
            pl.BlockSpec((2, H), lambda i: (0, 0)),
            pl.BlockSpec((1, H), lambda i: (0, 0)),
            pl.BlockSpec((1, H), lambda i: (0, 0)),
        ],
        out_specs=pl.BlockSpec((1, s, H), lambda i: (i, 0, 0)),
        out_shape=jax.ShapeDtypeStruct((b, s, H), jnp.float32),
    )(g3, tt3, pos_emb, type_emb, ln_w.reshape(1, H), ln_b.reshape(1, H))


def kernel(input_ids, token_type_ids, word_emb, pos_emb, type_emb, ln_w, ln_b):
    b, s = input_ids.shape
    assert word_emb.shape[1] == H
    ids = input_ids.reshape(-1).astype(jnp.int32)
    tts = token_type_ids.astype(jnp.int32)
    gath = _sc_gather(ids, word_emb)
    return _tc_ln(gath, tts, pos_emb, type_emb, ln_w, ln_b)
